# baseline (speedup 1.0000x reference)
import functools

import jax
import jax.numpy as jnp
from jax import lax
from jax.experimental import pallas as pl
from jax.experimental.pallas import tpu as pltpu
from jax.experimental.pallas import tpu_sc as plsc

N_ROWS = 16384
NC = 1000
VAL = 1001.0
NUM_WORKERS = 32
ROWS_PER_WORKER = N_ROWS // NUM_WORKERS
CHUNK_ROWS = 64
CHUNK_WORDS = CHUNK_ROWS * NC
N_CHUNKS = ROWS_PER_WORKER // CHUNK_ROWS


def _fill_ones(buf):
    ones16 = jnp.full((16,), 1.0, jnp.float32)

    def body(i, _):
        buf[pl.ds(i * 16, 16)] = ones16
        return 0

    lax.fori_loop(0, CHUNK_WORDS // 16, body, 0)


def _poke(buf, lab, chunk, value):
    iota = lax.iota(jnp.int32, 16)
    vals = jnp.full((16,), value, jnp.float32)
    for v in range(CHUNK_ROWS // 16):
        labv = lab[pl.ds(chunk * CHUNK_ROWS + v * 16, 16)]
        pos = (iota + v * 16) * NC + labv
        plsc.store_scatter(buf, [pos], vals)


def _sc_body(labels_hbm, out_hbm, buf0, buf1, lab, sem0, sem1):
    wid = lax.axis_index("s") * 2 + lax.axis_index("c")
    row0 = pl.multiple_of(wid * ROWS_PER_WORKER, 8)
    base = pl.multiple_of(wid * (ROWS_PER_WORKER * NC), 8)

    pltpu.sync_copy(labels_hbm.at[pl.ds(row0, ROWS_PER_WORKER)], lab)

    bufs = (buf0, buf1)
    sems = (sem0, sem1)
    copies = [None, None]

    for k in range(N_CHUNKS):
        b = k % 2
        if k < 2:
            _fill_ones(bufs[b])
        else:
            copies[b].wait()
            _poke(bufs[b], lab, k - 2, 1.0)
        _poke(bufs[b], lab, k, VAL)
        dst = out_hbm.at[pl.ds(pl.multiple_of(base + k * CHUNK_WORDS, 8),
                               CHUNK_WORDS)]
        copies[b] = pltpu.async_copy(bufs[b], dst, sems[b])

    copies[0].wait()
    copies[1].wait()


@functools.partial(
    pl.kernel,
    out_type=jax.ShapeDtypeStruct((N_ROWS * NC,), jnp.float32),
    mesh=plsc.VectorSubcoreMesh(core_axis_name="c", subcore_axis_name="s"),
    compiler_params=pltpu.CompilerParams(needs_layout_passes=False),
    scratch_types=[
        pltpu.VMEM((CHUNK_WORDS,), jnp.float32),
        pltpu.VMEM((CHUNK_WORDS,), jnp.float32),
        pltpu.VMEM((ROWS_PER_WORKER,), jnp.int32),
        pltpu.SemaphoreType.DMA,
        pltpu.SemaphoreType.DMA,
    ],
)
def _sc_smooth_onehot(labels_hbm, out_hbm, buf0, buf1, lab, sem0, sem1):
    _sc_body(labels_hbm, out_hbm, buf0, buf1, lab, sem0, sem1)


def kernel(labels):
    flat = _sc_smooth_onehot(labels.astype(jnp.int32))
    return flat.reshape(N_ROWS, NC)


# speedup vs baseline: 1.7199x; 1.7199x over previous
import functools

import jax
import jax.numpy as jnp
from jax import lax
from jax.experimental import pallas as pl
from jax.experimental.pallas import tpu as pltpu
from jax.experimental.pallas import tpu_sc as plsc

N_ROWS = 16384
NC = 1000
VAL = 1001.0
NUM_WORKERS = 32
ROWS_PER_WORKER = N_ROWS // NUM_WORKERS
CHUNK_ROWS = 32
N_CHUNKS = ROWS_PER_WORKER // CHUNK_ROWS


def _fill_ones(buf):
    ones16 = jnp.full((16,), 1.0, jnp.float32)

    def row_body(r, _):
        def col_body(c, _):
            buf[r, pl.ds(c * 16, 16)] = ones16
            return 0

        lax.fori_loop(0, NC // 16, col_body, 0)
        buf[r, pl.ds(NC - 16, 16)] = ones16
        return 0

    lax.fori_loop(0, CHUNK_ROWS, row_body, 0)


def _poke(buf, lab, chunk, value):
    iota = lax.iota(jnp.int32, 16)
    vals = jnp.full((16,), value, jnp.float32)
    for v in range(CHUNK_ROWS // 16):
        labv = lab[pl.ds(chunk * CHUNK_ROWS + v * 16, 16)]
        plsc.store_scatter(buf, [iota + v * 16, labv], vals)


@functools.partial(
    pl.kernel,
    out_type=jax.ShapeDtypeStruct((N_ROWS, NC), jnp.float32),
    mesh=plsc.VectorSubcoreMesh(core_axis_name="c", subcore_axis_name="s"),
    compiler_params=pltpu.CompilerParams(needs_layout_passes=False),
    scratch_types=[
        pltpu.VMEM((CHUNK_ROWS, NC), jnp.float32),
        pltpu.VMEM((CHUNK_ROWS, NC), jnp.float32),
        pltpu.VMEM((ROWS_PER_WORKER,), jnp.int32),
        pltpu.SemaphoreType.DMA,
        pltpu.SemaphoreType.DMA,
    ],
)
def _sc_smooth_onehot(labels_hbm, out_hbm, buf0, buf1, lab, sem0, sem1):
    wid = lax.axis_index("s") * 2 + lax.axis_index("c")
    row0 = pl.multiple_of(wid * ROWS_PER_WORKER, 8)

    pltpu.sync_copy(labels_hbm.at[pl.ds(row0, ROWS_PER_WORKER)], lab)

    bufs = (buf0, buf1)
    sems = (sem0, sem1)
    copies = [None, None]

    for k in range(N_CHUNKS):
        b = k % 2
        if k < 2:
            _fill_ones(bufs[b])
        else:
            copies[b].wait()
            _poke(bufs[b], lab, k - 2, 1.0)
        _poke(bufs[b], lab, k, VAL)
        dst = out_hbm.at[pl.ds(pl.multiple_of(row0 + k * CHUNK_ROWS, 8),
                               CHUNK_ROWS), :]
        copies[b] = pltpu.async_copy(bufs[b], dst, sems[b])

    copies[0].wait()
    copies[1].wait()


def kernel(labels):
    return _sc_smooth_onehot(labels.astype(jnp.int32))


# speedup vs baseline: 2.1760x; 1.2652x over previous
import jax
import jax.numpy as jnp
from jax.experimental import pallas as pl
from jax.experimental.pallas import tpu as pltpu

N_ROWS = 16384
NC = 1000
VAL = 1001.0
CHUNK = 1024
NBUF = 4
NCHUNKS = N_ROWS // CHUNK


def _tc_body(lab_ref, out_ref, b0, b1, b2, b3, s0, s1, s2, s3):
    bufs = (b0, b1, b2, b3)
    sems = (s0, s1, s2, s3)
    copies = [None] * NBUF
    for k in range(NCHUNKS):
        s = k % NBUF
        if k >= NBUF:
            copies[s].wait()
        lab = lab_ref[pl.ds(k * CHUNK, CHUNK), :]
        col = jax.lax.broadcasted_iota(jnp.int32, (CHUNK, NC), 1)
        bufs[s][...] = jnp.where(lab == col, VAL, 1.0)
        copies[s] = pltpu.make_async_copy(
            bufs[s], out_ref.at[pl.ds(k * CHUNK, CHUNK), :], sems[s])
        copies[s].start()
    for k in range(NCHUNKS - NBUF, NCHUNKS):
        copies[k % NBUF].wait()


def kernel(labels):
    lab2d = labels.astype(jnp.int32).reshape(N_ROWS, 1)
    return pl.pallas_call(
        _tc_body,
        in_specs=[pl.BlockSpec(memory_space=pltpu.VMEM)],
        out_specs=pl.BlockSpec(memory_space=pltpu.HBM),
        out_shape=jax.ShapeDtypeStruct((N_ROWS, NC), jnp.float32),
        scratch_shapes=(
            [pltpu.VMEM((CHUNK, NC), jnp.float32)] * NBUF
            + [pltpu.SemaphoreType.DMA] * NBUF
        ),
    )(lab2d)
